# Initial kernel scaffold; baseline (speedup 1.0000x reference)
import jax
import jax.numpy as jnp
from jax.experimental import pallas as pl


def kernel(points, depth_map, pose, K, descriptors, env, memory_table):
    raise NotImplementedError("write your pallas kernel here")


# speedup vs baseline: 1.8532x; 1.8532x over previous
import functools

import jax
import jax.numpy as jnp
from jax import lax
from jax.experimental import pallas as pl
from jax.experimental.pallas import tpu as pltpu
from jax.experimental.pallas import tpu_sc as plsc

N_MEM = 50000
NPTS = 65536
D = 256
NW = 32
PPW = NPTS // NW
CH1 = 128
NCH1 = PPW // CH1
B2 = 64
NCH2 = PPW // B2
FMAX = 3.4028235e38

_mesh = plsc.VectorSubcoreMesh(core_axis_name="c", subcore_axis_name="s",
                               num_cores=2, num_subcores=16)


def _wid():
    return lax.axis_index("s") * 2 + lax.axis_index("c")


def _floor_i32(x):
    t = x.astype(jnp.int32)
    return t - (x < t.astype(jnp.float32)).astype(jnp.int32)


def _rsqrt16(x):
    i = lax.bitcast_convert_type(x, jnp.int32)
    i = jnp.int32(0x5F3759DF) - lax.shift_right_arithmetic(i, 1)
    y = lax.bitcast_convert_type(i, jnp.float32)
    for _ in range(4):
        y = y * (jnp.float32(1.5) - jnp.float32(0.5) * x * y * y)
    return y


@functools.partial(
    pl.kernel,
    out_type=[
        jax.ShapeDtypeStruct((NPTS,), jnp.int32),
        jax.ShapeDtypeStruct((NPTS,), jnp.float32),
        jax.ShapeDtypeStruct((N_MEM,), jnp.int32),
    ],
    mesh=_mesh,
    compiler_params=pltpu.CompilerParams(needs_layout_passes=False),
    scratch_types=[
        pltpu.VMEM((PPW,), jnp.float32),
        pltpu.VMEM((PPW,), jnp.float32),
        pltpu.VMEM((NCH1, CH1), jnp.int32),
        pltpu.VMEM((NCH1, CH1), jnp.float32),
        pltpu.VMEM((NCH1, CH1), jnp.int32),
        pltpu.VMEM((NCH1, CH1), jnp.float32),
        pltpu.VMEM((NCH1, CH1), jnp.int32),
        pltpu.VMEM((16,), jnp.float32),
        pltpu.VMEM((16,), jnp.float32),
        pltpu.SemaphoreType.DMA,
        pltpu.SemaphoreType.DMA,
    ],
)
def _stage1(u_hbm, v_hbm, depth_hbm, pose_hbm, kmat_hbm,
            idx_hbm, w_hbm, winner_hbm,
            u_v, v_v, pix_v, d_v, idx_v, w_v, myid_v, pose_v, k_v,
            dsem, ssem):
    wid = _wid()
    b = wid // 4
    base = wid * PPW

    pltpu.sync_copy(u_hbm.at[pl.ds(base, PPW)], u_v)
    pltpu.sync_copy(v_hbm.at[pl.ds(base, PPW)], v_v)
    pltpu.sync_copy(pose_hbm.at[b], pose_v)
    pltpu.sync_copy(kmat_hbm.at[b], k_v)

    kk = k_v[pl.ds(0, 16)]
    pp = pose_v[pl.ds(0, 16)]
    fx = kk[0]
    cx = kk[2]
    fy = kk[4]
    cy = kk[5]
    r00, r01, r02, t0 = pp[0], pp[1], pp[2], pp[3]
    r10, r11, r12, t1 = pp[4], pp[5], pp[6], pp[7]
    r20, r21, r22, t2 = pp[8], pp[9], pp[10], pp[11]
    pixbase = b * (480 * 640)
    iota16 = lax.iota(jnp.int32, 16)

    def chunk(j, _):
        off0 = j * CH1
        for k in range(CH1 // 16):
            off = pl.multiple_of(off0 + k * 16, 16)
            u = u_v[pl.ds(off, 16)] * jnp.float32(639.0)
            v = v_v[pl.ds(off, 16)] * jnp.float32(479.0)
            ui = jnp.minimum(jnp.maximum((u + jnp.float32(0.5)).astype(jnp.int32), 0), 639)
            vi = jnp.minimum(jnp.maximum((v + jnp.float32(0.5)).astype(jnp.int32), 0), 479)
            pix_v[j, pl.ds(k * 16, 16)] = pixbase + vi * 640 + ui
        pltpu.async_copy(depth_hbm.at[pix_v.at[j]], d_v.at[j], dsem).wait()
        for k in range(CH1 // 16):
            off = pl.multiple_of(off0 + k * 16, 16)
            u = u_v[pl.ds(off, 16)] * jnp.float32(639.0)
            v = v_v[pl.ds(off, 16)] * jnp.float32(479.0)
            d = d_v[j, pl.ds(k * 16, 16)]
            xc = (u - cx) / fx * d
            yc = (v - cy) / fy * d
            wx = r00 * xc + r01 * yc + r02 * d + t0
            wy = r10 * xc + r11 * yc + r12 * d + t1
            wz = r20 * xc + r21 * yc + r22 * d + t2
            ci = _floor_i32(wx * jnp.float32(4.0))
            cj = _floor_i32(wy * jnp.float32(4.0))
            ck = _floor_i32(wz * jnp.float32(4.0))
            h = ((ci * jnp.int32(73856093))
                 ^ (cj * jnp.int32(19349669))
                 ^ (ck * jnp.int32(83492791)))
            m = lax.rem(h, jnp.int32(N_MEM))
            slot = jnp.where(m < 0, m + jnp.int32(N_MEM), m)
            fmax = jnp.float32(FMAX)
            fin = ((jnp.abs(wx) <= fmax)
                   & (jnp.abs(wy) <= fmax)
                   & (jnp.abs(wz) <= fmax))
            idx_v[j, pl.ds(k * 16, 16)] = slot
            w_v[j, pl.ds(k * 16, 16)] = fin.astype(jnp.float32)
            myid_v[j, pl.ds(k * 16, 16)] = base + off0 + k * 16 + iota16
        pltpu.async_copy(myid_v.at[j], winner_hbm.at[idx_v.at[j]], ssem).wait()
        pltpu.sync_copy(idx_v.at[j], idx_hbm.at[pl.ds(base + off0, CH1)])
        pltpu.sync_copy(w_v.at[j], w_hbm.at[pl.ds(base + off0, CH1)])
        return 0

    lax.fori_loop(0, NCH1, chunk, 0)


@functools.partial(
    pl.kernel,
    out_type=jax.ShapeDtypeStruct((NW, 2, 16), jnp.float32),
    mesh=_mesh,
    compiler_params=pltpu.CompilerParams(needs_layout_passes=False),
    scratch_types=[
        pltpu.VMEM((PPW,), jnp.int32),
        pltpu.VMEM((PPW,), jnp.int32),
        pltpu.VMEM((PPW,), jnp.float32),
        pltpu.VMEM((B2, D), jnp.float32),
        pltpu.VMEM((B2, D), jnp.float32),
        pltpu.VMEM((B2, D), jnp.float32),
        pltpu.VMEM((16,), jnp.float32),
        pltpu.VMEM((16,), jnp.float32),
        pltpu.SemaphoreType.DMA,
        pltpu.SemaphoreType.DMA,
        pltpu.SemaphoreType.DMA,
    ],
)
def _stage2(idx_hbm, winner_hbm, desc_hbm, mem_hbm, w_hbm,
            acc_hbm,
            idx_v, wptr_v, w_v, desc_b, old_b, wd_b,
            accc_v, accw_v, sem1, sem2, sem3):
    wid = _wid()
    base = wid * PPW

    pltpu.sync_copy(idx_hbm.at[pl.ds(base, PPW)], idx_v)
    pltpu.sync_copy(w_hbm.at[pl.ds(base, PPW)], w_v)

    def resolve(wt_v):
        pltpu.sync_copy(winner_hbm, wt_v)

        def gj(j, _):
            off = pl.multiple_of(j * 16, 16)
            wptr_v[pl.ds(off, 16)] = plsc.load_gather(wt_v, [idx_v[pl.ds(off, 16)]])
            return 0

        lax.fori_loop(0, PPW // 16, gj, 0)

    pl.run_scoped(resolve, pltpu.VMEM((N_MEM,), jnp.int32))

    accc_v[...] = jnp.zeros((16,), jnp.float32)
    accw_v[...] = jnp.zeros((16,), jnp.float32)

    def chunk(c, _):
        cb = pl.multiple_of(c * B2, 64)
        h1 = pltpu.async_copy(desc_hbm.at[pl.ds(base + cb, B2)], desc_b, sem1)
        h2 = pltpu.async_copy(mem_hbm.at[idx_v.at[pl.ds(cb, B2)]], old_b, sem2)
        h3 = pltpu.async_copy(desc_hbm.at[wptr_v.at[pl.ds(cb, B2)]], wd_b, sem3)
        h1.wait()
        h2.wait()
        h3.wait()

        iota16 = lax.iota(jnp.int32, 16)
        for q in range(B2 // 16):
            def point(p16, carry):
                s1a, s2a, s3a = carry
                p = q * 16 + p16
                a1 = jnp.zeros((16,), jnp.float32)
                a2 = jnp.zeros((16,), jnp.float32)
                a3 = jnp.zeros((16,), jnp.float32)
                for k in range(D // 16):
                    a = desc_b[p, pl.ds(k * 16, 16)]
                    o = old_b[p, pl.ds(k * 16, 16)]
                    g = wd_b[p, pl.ds(k * 16, 16)]
                    m = o + g
                    a1 = a1 + a * m
                    a2 = a2 + m * m
                    a3 = a3 + a * a
                lane = iota16 == p16
                s1a = jnp.where(lane, jnp.full((16,), jnp.sum(a1), jnp.float32), s1a)
                s2a = jnp.where(lane, jnp.full((16,), jnp.sum(a2), jnp.float32), s2a)
                s3a = jnp.where(lane, jnp.full((16,), jnp.sum(a3), jnp.float32), s3a)
                return s1a, s2a, s3a

            z16 = jnp.zeros((16,), jnp.float32)
            s1, s2, s3 = lax.fori_loop(0, 16, point, (z16, z16, z16))
            w16 = w_v[pl.ds(cb + q * 16, 16)]
            den = jnp.maximum(s2 * s3, jnp.float32(1e-28))
            cos = s1 * _rsqrt16(den)
            accc_v[...] = accc_v[...] + cos * w16
            accw_v[...] = accw_v[...] + w16
        return 0

    lax.fori_loop(0, NCH2, chunk, 0)

    pltpu.sync_copy(accc_v, acc_hbm.at[wid, 0])
    pltpu.sync_copy(accw_v, acc_hbm.at[wid, 1])


def kernel(points, depth_map, pose, K, descriptors, env, memory_table):
    B = depth_map.shape[0]
    u_in = points[..., 0].reshape(-1)
    v_in = points[..., 1].reshape(-1)
    depth_flat = depth_map.reshape(-1)
    pose_flat = pose.reshape(B, 16)
    k_pad = jnp.zeros((B, 16), jnp.float32).at[:, :9].set(K.reshape(B, 9))
    desc_f = descriptors.reshape(-1, D)

    idx, w, winner = _stage1(u_in, v_in, depth_flat, pose_flat, k_pad)
    acc = _stage2(idx, winner, desc_f, memory_table, w)
    sc = jnp.sum(acc[:, 0, :])
    sw = jnp.sum(acc[:, 1, :])
    return (jnp.float32(1.0) - sc / jnp.maximum(sw, jnp.float32(1.0))).astype(jnp.float32)


# speedup vs baseline: 2.3751x; 1.2816x over previous
import functools

import jax
import jax.numpy as jnp
from jax import lax
from jax.experimental import pallas as pl
from jax.experimental.pallas import tpu as pltpu
from jax.experimental.pallas import tpu_sc as plsc

N_MEM = 50000
NPTS = 65536
D = 256
NW = 32
PPW = NPTS // NW
CH1 = 128
NCH1 = PPW // CH1
B2 = 64
NCH2 = PPW // B2
FMAX = 3.4028235e38

_mesh = plsc.VectorSubcoreMesh(core_axis_name="c", subcore_axis_name="s",
                               num_cores=2, num_subcores=16)
_params = pltpu.CompilerParams(needs_layout_passes=False)


def _wid():
    return lax.axis_index("s") * 2 + lax.axis_index("c")


def _floor_i32(x):
    t = x.astype(jnp.int32)
    return t - (x < t.astype(jnp.float32)).astype(jnp.int32)


def _rsqrt16(x):
    i = lax.bitcast_convert_type(x, jnp.int32)
    i = jnp.int32(0x5F3759DF) - lax.shift_right_arithmetic(i, 1)
    y = lax.bitcast_convert_type(i, jnp.float32)
    for _ in range(4):
        y = y * (jnp.float32(1.5) - jnp.float32(0.5) * x * y * y)
    return y


@functools.partial(
    pl.kernel,
    out_type=[
        jax.ShapeDtypeStruct((NPTS,), jnp.int32),
        jax.ShapeDtypeStruct((NPTS,), jnp.float32),
        jax.ShapeDtypeStruct((N_MEM,), jnp.int32),
    ],
    mesh=_mesh,
    compiler_params=_params,
    scratch_types=[
        pltpu.VMEM((PPW,), jnp.float32),
        pltpu.VMEM((PPW,), jnp.float32),
        pltpu.VMEM((NCH1, CH1), jnp.int32),
        pltpu.VMEM((NCH1, CH1), jnp.float32),
        pltpu.VMEM((NCH1, CH1), jnp.int32),
        pltpu.VMEM((NCH1, CH1), jnp.float32),
        pltpu.VMEM((NCH1, CH1), jnp.int32),
        pltpu.VMEM((16,), jnp.float32),
        pltpu.VMEM((16,), jnp.float32),
        pltpu.SemaphoreType.DMA,
        pltpu.SemaphoreType.DMA,
        pltpu.SemaphoreType.DMA,
    ],
)
def _stage1(u_hbm, v_hbm, depth_hbm, pose_hbm, kmat_hbm,
            idx_hbm, w_hbm, winner_hbm,
            u_v, v_v, pix_v, d_v, idx_v, w_v, myid_v, pose_v, k_v,
            gsemA, gsemB, ssem):
    wid = _wid()
    b = wid // 4
    base = wid * PPW

    pltpu.sync_copy(u_hbm.at[pl.ds(base, PPW)], u_v)
    pltpu.sync_copy(v_hbm.at[pl.ds(base, PPW)], v_v)
    pltpu.sync_copy(pose_hbm.at[b], pose_v)
    pltpu.sync_copy(kmat_hbm.at[b], k_v)

    kk = k_v[pl.ds(0, 16)]
    pp = pose_v[pl.ds(0, 16)]
    fx = kk[0]
    cx = kk[2]
    fy = kk[4]
    cy = kk[5]
    r00, r01, r02, t0 = pp[0], pp[1], pp[2], pp[3]
    r10, r11, r12, t1 = pp[4], pp[5], pp[6], pp[7]
    r20, r21, r22, t2 = pp[8], pp[9], pp[10], pp[11]
    pixbase = b * (480 * 640)
    iota16 = lax.iota(jnp.int32, 16)

    def fill_pix(j):
        off0 = j * CH1
        for k in range(CH1 // 16):
            off = pl.multiple_of(off0 + k * 16, 16)
            u = u_v[pl.ds(off, 16)] * jnp.float32(639.0)
            v = v_v[pl.ds(off, 16)] * jnp.float32(479.0)
            ui = jnp.minimum(jnp.maximum((u + jnp.float32(0.5)).astype(jnp.int32), 0), 639)
            vi = jnp.minimum(jnp.maximum((v + jnp.float32(0.5)).astype(jnp.int32), 0), 479)
            pix_v[j, pl.ds(k * 16, 16)] = pixbase + vi * 640 + ui

    def issue_gather(j, sem):
        return pltpu.async_copy(depth_hbm.at[pix_v.at[j]], d_v.at[j], sem)

    fill_pix(0)
    issue_gather(0, gsemA)
    fill_pix(1)
    issue_gather(1, gsemB)

    def chunk(j, _):
        off0 = j * CH1

        @pl.when(lax.rem(j, 2) == 0)
        def _():
            pltpu.make_async_copy(depth_hbm.at[pix_v.at[j]], d_v.at[j], gsemA).wait()

        @pl.when(lax.rem(j, 2) == 1)
        def _():
            pltpu.make_async_copy(depth_hbm.at[pix_v.at[j]], d_v.at[j], gsemB).wait()

        for k in range(CH1 // 16):
            off = pl.multiple_of(off0 + k * 16, 16)
            u = u_v[pl.ds(off, 16)] * jnp.float32(639.0)
            v = v_v[pl.ds(off, 16)] * jnp.float32(479.0)
            d = d_v[j, pl.ds(k * 16, 16)]
            xc = (u - cx) / fx * d
            yc = (v - cy) / fy * d
            wx = r00 * xc + r01 * yc + r02 * d + t0
            wy = r10 * xc + r11 * yc + r12 * d + t1
            wz = r20 * xc + r21 * yc + r22 * d + t2
            ci = _floor_i32(wx * jnp.float32(4.0))
            cj = _floor_i32(wy * jnp.float32(4.0))
            ck = _floor_i32(wz * jnp.float32(4.0))
            h = ((ci * jnp.int32(73856093))
                 ^ (cj * jnp.int32(19349669))
                 ^ (ck * jnp.int32(83492791)))
            m = lax.rem(h, jnp.int32(N_MEM))
            slot = jnp.where(m < 0, m + jnp.int32(N_MEM), m)
            fmax = jnp.float32(FMAX)
            fin = ((jnp.abs(wx) <= fmax)
                   & (jnp.abs(wy) <= fmax)
                   & (jnp.abs(wz) <= fmax))
            idx_v[j, pl.ds(k * 16, 16)] = slot
            w_v[j, pl.ds(k * 16, 16)] = fin.astype(jnp.float32)
            myid_v[j, pl.ds(k * 16, 16)] = base + off0 + k * 16 + iota16

        pltpu.async_copy(myid_v.at[j], winner_hbm.at[idx_v.at[j]], ssem)

        @pl.when(j + 2 < NCH1)
        def _():
            fill_pix(j + 2)

            @pl.when(lax.rem(j, 2) == 0)
            def _():
                issue_gather(j + 2, gsemA)

            @pl.when(lax.rem(j, 2) == 1)
            def _():
                issue_gather(j + 2, gsemB)

        return 0

    lax.fori_loop(0, NCH1, chunk, 0)

    def out_row(j, _):
        pltpu.async_copy(idx_v.at[j], idx_hbm.at[pl.ds(base + j * CH1, CH1)], ssem)
        pltpu.async_copy(w_v.at[j], w_hbm.at[pl.ds(base + j * CH1, CH1)], ssem)
        return 0

    lax.fori_loop(0, NCH1, out_row, 0)

    def drain(j, _):
        pltpu.make_async_copy(myid_v.at[j], winner_hbm.at[idx_v.at[j]], ssem).wait()
        pltpu.make_async_copy(idx_v.at[j], idx_hbm.at[pl.ds(base + j * CH1, CH1)], ssem).wait()
        pltpu.make_async_copy(w_v.at[j], w_hbm.at[pl.ds(base + j * CH1, CH1)], ssem).wait()
        return 0

    lax.fori_loop(0, NCH1, drain, 0)


@functools.partial(
    pl.kernel,
    out_type=jax.ShapeDtypeStruct((NW, 2, 16), jnp.float32),
    mesh=_mesh,
    compiler_params=_params,
    scratch_types=[
        pltpu.VMEM((PPW,), jnp.int32),
        pltpu.VMEM((PPW,), jnp.int32),
        pltpu.VMEM((PPW,), jnp.float32),
        pltpu.VMEM((16,), jnp.float32),
        pltpu.VMEM((16,), jnp.float32),
        pltpu.SemaphoreType.DMA,
        pltpu.SemaphoreType.DMA,
    ],
)
def _stage2(idx_hbm, winner_hbm, desc_hbm, mem_hbm, w_hbm,
            acc_hbm,
            idx_v, wptr_v, w_v, accc_v, accw_v, semA, semB):
    wid = _wid()
    base = wid * PPW

    pltpu.sync_copy(idx_hbm.at[pl.ds(base, PPW)], idx_v)
    pltpu.sync_copy(w_hbm.at[pl.ds(base, PPW)], w_v)

    def resolve(wt_v):
        pltpu.sync_copy(winner_hbm, wt_v)

        def gj(j, _):
            off = pl.multiple_of(j * 16, 16)
            wptr_v[pl.ds(off, 16)] = plsc.load_gather(wt_v, [idx_v[pl.ds(off, 16)]])
            return 0

        lax.fori_loop(0, PPW // 16, gj, 0)

    pl.run_scoped(resolve, pltpu.VMEM((N_MEM,), jnp.int32))

    accc_v[...] = jnp.zeros((16,), jnp.float32)
    accw_v[...] = jnp.zeros((16,), jnp.float32)

    def mainloop(desc_b0, old_b0, wd_b0, desc_b1, old_b1, wd_b1):
        bufs = ((desc_b0, old_b0, wd_b0), (desc_b1, old_b1, wd_b1))

        def issue(c, par):
            db, ob, wb = bufs[par]
            sem = (semA, semB)[par]
            cb = pl.multiple_of(c * B2, 64)
            pltpu.async_copy(desc_hbm.at[pl.ds(base + cb, B2)], db, sem)
            pltpu.async_copy(mem_hbm.at[idx_v.at[pl.ds(cb, B2)]], ob, sem)
            pltpu.async_copy(desc_hbm.at[wptr_v.at[pl.ds(cb, B2)]], wb, sem)

        def wait(c, par):
            db, ob, wb = bufs[par]
            sem = (semA, semB)[par]
            cb = pl.multiple_of(c * B2, 64)
            pltpu.make_async_copy(desc_hbm.at[pl.ds(base + cb, B2)], db, sem).wait()
            pltpu.make_async_copy(mem_hbm.at[idx_v.at[pl.ds(cb, B2)]], ob, sem).wait()
            pltpu.make_async_copy(desc_hbm.at[wptr_v.at[pl.ds(cb, B2)]], wb, sem).wait()

        issue(0, 0)
        issue(1, 1)
        iota16 = lax.iota(jnp.int32, 16)

        def compute(c, par):
            db, ob, wb = bufs[par]
            cb = pl.multiple_of(c * B2, 64)
            for q in range(B2 // 16):
                def point(p16, carry):
                    s1a, s2a, s3a = carry
                    p = q * 16 + p16
                    a1 = jnp.zeros((16,), jnp.float32)
                    a2 = jnp.zeros((16,), jnp.float32)
                    a3 = jnp.zeros((16,), jnp.float32)
                    for k in range(D // 16):
                        a = db[p, pl.ds(k * 16, 16)]
                        o = ob[p, pl.ds(k * 16, 16)]
                        g = wb[p, pl.ds(k * 16, 16)]
                        m = o + g
                        a1 = a1 + a * m
                        a2 = a2 + m * m
                        a3 = a3 + a * a
                    lane = iota16 == p16
                    s1a = jnp.where(lane, jnp.full((16,), jnp.sum(a1), jnp.float32), s1a)
                    s2a = jnp.where(lane, jnp.full((16,), jnp.sum(a2), jnp.float32), s2a)
                    s3a = jnp.where(lane, jnp.full((16,), jnp.sum(a3), jnp.float32), s3a)
                    return s1a, s2a, s3a

                z16 = jnp.zeros((16,), jnp.float32)
                s1, s2, s3 = lax.fori_loop(0, 16, point, (z16, z16, z16))
                w16 = w_v[pl.ds(cb + q * 16, 16)]
                den = jnp.maximum(s2 * s3, jnp.float32(1e-28))
                cos = s1 * _rsqrt16(den)
                accc_v[...] = accc_v[...] + cos * w16
                accw_v[...] = accw_v[...] + w16

        def chunk(c, _):
            @pl.when(lax.rem(c, 2) == 0)
            def _():
                wait(c, 0)

            @pl.when(lax.rem(c, 2) == 1)
            def _():
                wait(c, 1)

            @pl.when(lax.rem(c, 2) == 0)
            def _():
                compute(c, 0)

            @pl.when(lax.rem(c, 2) == 1)
            def _():
                compute(c, 1)

            @pl.when(c + 2 < NCH2)
            def _():
                @pl.when(lax.rem(c, 2) == 0)
                def _():
                    issue(c + 2, 0)

                @pl.when(lax.rem(c, 2) == 1)
                def _():
                    issue(c + 2, 1)

            return 0

        lax.fori_loop(0, NCH2, chunk, 0)

    pl.run_scoped(
        mainloop,
        pltpu.VMEM((B2, D), jnp.float32), pltpu.VMEM((B2, D), jnp.float32),
        pltpu.VMEM((B2, D), jnp.float32), pltpu.VMEM((B2, D), jnp.float32),
        pltpu.VMEM((B2, D), jnp.float32), pltpu.VMEM((B2, D), jnp.float32),
    )

    pltpu.sync_copy(accc_v, acc_hbm.at[wid, 0])
    pltpu.sync_copy(accw_v, acc_hbm.at[wid, 1])


def kernel(points, depth_map, pose, K, descriptors, env, memory_table):
    B = depth_map.shape[0]
    u_in = points[..., 0].reshape(-1)
    v_in = points[..., 1].reshape(-1)
    depth_flat = depth_map.reshape(-1)
    pose_flat = pose.reshape(B, 16)
    k_pad = jnp.zeros((B, 16), jnp.float32).at[:, :9].set(K.reshape(B, 9))
    desc_f = descriptors.reshape(-1, D)

    idx, w, winner = _stage1(u_in, v_in, depth_flat, pose_flat, k_pad)
    acc = _stage2(idx, winner, desc_f, memory_table, w)
    sc = jnp.sum(acc[:, 0, :])
    sw = jnp.sum(acc[:, 1, :])
    return (jnp.float32(1.0) - sc / jnp.maximum(sw, jnp.float32(1.0))).astype(jnp.float32)
